# Initial kernel scaffold; baseline (speedup 1.0000x reference)
import jax
import jax.numpy as jnp
from jax.experimental import pallas as pl


def kernel(hidden_states, topk_indices, topk_weights, gate_up_proj, down_proj):
    raise NotImplementedError("write your pallas kernel here")


# speedup vs baseline: 1.3057x; 1.3057x over previous
import jax
import jax.numpy as jnp
from jax.experimental import pallas as pl
from jax.experimental.pallas import tpu as pltpu

E = 8
D_MODEL = 2048
D_FF = 1024
T = 2048
TOP_K = 2
BT = 256


def _moe_dense_body(idx_ref, wts_ref, x_ref, gu_ref, dp_ref, out_ref):
    e = pl.program_id(0)
    i = pl.program_id(1)
    x = x_ref[...].astype(jnp.bfloat16)
    gate = jax.lax.dot_general(
        x, gu_ref[0, :D_FF, :],
        (((1,), (1,)), ((), ())),
        preferred_element_type=jnp.float32)
    up = jax.lax.dot_general(
        x, gu_ref[0, D_FF:, :],
        (((1,), (1,)), ((), ())),
        preferred_element_type=jnp.float32)
    h = (jax.nn.silu(gate) * up).astype(jnp.bfloat16)
    eo = jax.lax.dot_general(
        h, dp_ref[0],
        (((1,), (1,)), ((), ())),
        preferred_element_type=jnp.float32)
    w = jnp.sum(jnp.where(idx_ref[...] == e, wts_ref[...], 0.0), axis=1)
    contrib = w[:, None] * eo
    row = pl.ds(i * BT, BT)

    @pl.when(e == 0)
    def _():
        out_ref[row, :] = contrib

    @pl.when(e != 0)
    def _():
        out_ref[row, :] = out_ref[row, :] + contrib


def kernel(hidden_states, topk_indices, topk_weights, gate_up_proj, down_proj):
    idx = topk_indices.astype(jnp.int32)
    gate_up_proj = gate_up_proj.astype(jnp.bfloat16)
    down_proj = down_proj.astype(jnp.bfloat16)
    grid = (E, T // BT)
    return pl.pallas_call(
        _moe_dense_body,
        grid=grid,
        in_specs=[
            pl.BlockSpec((BT, TOP_K), lambda e, i: (i, 0)),
            pl.BlockSpec((BT, TOP_K), lambda e, i: (i, 0)),
            pl.BlockSpec((BT, D_MODEL), lambda e, i: (i, 0)),
            pl.BlockSpec((1, 2 * D_FF, D_MODEL), lambda e, i: (e, 0, 0)),
            pl.BlockSpec((1, D_MODEL, D_FF), lambda e, i: (e, 0, 0)),
        ],
        out_specs=pl.BlockSpec((T, D_MODEL), lambda e, i: (0, 0)),
        out_shape=jax.ShapeDtypeStruct((T, D_MODEL), jnp.float32),
    )(idx, topk_weights, hidden_states, gate_up_proj, down_proj)
